# baseline (speedup 1.0000x reference)
import functools

import jax
import jax.numpy as jnp
from jax import lax
from jax.experimental import pallas as pl
from jax.experimental.pallas import tpu as pltpu
from jax.experimental.pallas import tpu_sc as plsc

B = 4096
LTBL = 1000000
D = 64
S = 200
LANES = 16
NC, NS = 2, 16
NW = NC * NS
BPW = B // NW
SPAD = 208
NBLK = SPAD // LANES
C0, C1 = 104, 96
D_UNROLL = 4


def _body(input_hbm, shorty_hbm, table_hbm, out_hbm,
          idx_v, in_v, rows0, rows1, out_v, sem0, sem1):
    wid = lax.axis_index("c") * NS + lax.axis_index("s")
    base = wid * BPW

    pltpu.sync_copy(shorty_hbm.at[pl.ds(base, BPW)], idx_v)
    pltpu.sync_copy(input_hbm.at[pl.ds(base, BPW)], in_v)

    def start_gather(i, rows, sem):
        pltpu.async_copy(table_hbm.at[idx_v.at[i, pl.ds(0, C0)]],
                         rows.at[pl.ds(0, C0)], sem)
        pltpu.async_copy(table_hbm.at[idx_v.at[i, pl.ds(C0, C1)]],
                         rows.at[pl.ds(C0, C1)], sem)

    def wait_gather(rows, sem):
        pltpu.make_async_copy(table_hbm.at[pl.ds(0, S)],
                              rows.at[pl.ds(0, S)], sem).wait()

    s_idx = [jnp.int32(k * LANES) + lax.iota(jnp.int32, LANES)
             for k in range(NBLK - 1)]
    s_idx.append(jnp.int32(SPAD - LANES) + lax.iota(jnp.int32, LANES))

    def compute(i, rows):
        zero = jnp.zeros((LANES,), jnp.float32)
        accs = (zero,) * NBLK
        for q in range(D // LANES):
            in_q = in_v[i, pl.ds(q * LANES, LANES)]

            def dstep(d2, accs_t, q=q, in_q=in_q):
                dv16 = jnp.full((LANES,), d2, jnp.int32)
                xb = in_q.at[dv16].get(
                    mode=lax.GatherScatterMode.PROMISE_IN_BOUNDS)
                dv = jnp.full((LANES,), q * LANES + d2, jnp.int32)
                return tuple(
                    accs_t[k] + plsc.load_gather(rows, [s_idx[k], dv]) * xb
                    for k in range(NBLK))

            accs = lax.fori_loop(0, LANES, dstep, accs, unroll=D_UNROLL)
        for k in range(NBLK):
            out_v[i, pl.ds(k * LANES, LANES)] = accs[k]

    start_gather(jnp.int32(0), rows0, sem0)

    def step(it, carry):
        g = it * 2
        start_gather(g + 1, rows1, sem1)
        wait_gather(rows0, sem0)
        compute(g, rows0)

        @pl.when(g + 2 < BPW)
        def _():
            start_gather(g + 2, rows0, sem0)

        wait_gather(rows1, sem1)
        compute(g + 1, rows1)
        return carry

    lax.fori_loop(0, BPW // 2, step, 0)
    pltpu.sync_copy(out_v, out_hbm.at[wid])


def kernel(input, lbl_ft, shorty, attn_w):
    w = jax.nn.softmax(attn_w)
    x = input * w[0]
    idx = shorty.astype(jnp.int32)
    mesh = plsc.VectorSubcoreMesh(core_axis_name="c", subcore_axis_name="s")
    run = pl.kernel(
        _body,
        out_type=jax.ShapeDtypeStruct((NW, BPW, SPAD), jnp.float32),
        mesh=mesh,
        scratch_types=[
            pltpu.VMEM((BPW, S), jnp.int32),
            pltpu.VMEM((BPW, D), jnp.float32),
            pltpu.VMEM((SPAD, D), jnp.float32),
            pltpu.VMEM((SPAD, D), jnp.float32),
            pltpu.VMEM((BPW, SPAD), jnp.float32),
            pltpu.SemaphoreType.DMA,
            pltpu.SemaphoreType.DMA,
        ],
        compiler_params=pltpu.CompilerParams(use_tc_tiling_on_sc=False,
                                             needs_layout_passes=False),
    )
    padded = run(x, idx, lbl_ft)
    return padded.reshape(B, SPAD)[:, :S]
